# Initial kernel scaffold; baseline (speedup 1.0000x reference)
import jax
import jax.numpy as jnp
from jax.experimental import pallas as pl


def kernel(x, Wg, bg, w1, w2):
    raise NotImplementedError("write your pallas kernel here")


# speedup vs baseline: 1.0403x; 1.0403x over previous
import functools

import jax
import jax.numpy as jnp
from jax.experimental import pallas as pl
from jax.experimental.pallas import tpu as pltpu

E = 8
K = 2
H = 2048
F = 1024
T = 2048
P = T * K
BM = 256
NB = P // BM


def _cumsum1_excl(a):
    n = a.shape[1]
    z = jnp.zeros((a.shape[0], 1), a.dtype)
    a = jnp.concatenate([z, a[:, :-1]], axis=1)
    s = 1
    while s < n:
        zz = jnp.zeros((a.shape[0], s), a.dtype)
        a = a + jnp.concatenate([zz, a[:, :-s]], axis=1)
        s *= 2
    return a


def _cumsum0(a):
    n = a.shape[0]
    z = jnp.zeros((1,) + a.shape[1:], a.dtype)
    a = jnp.concatenate([z, a[:-1]], axis=0)
    s = 1
    while s < n:
        zz = jnp.zeros((s,) + a.shape[1:], a.dtype)
        a = a + jnp.concatenate([zz, a[:-s]], axis=0)
        s *= 2
    return a


def _router_body(x_ref, wg_ref, bg_ref,
                 pos_ref, w_ref, off_ref, lo_ref, hi_ref):
    x = x_ref[...]
    wg = wg_ref[...]
    logits = jax.lax.dot_general(
        x, wg, (((1,), (1,)), ((), ())),
        preferred_element_type=jnp.float32)
    logits = logits + bg_ref[...]

    iota_e = jax.lax.broadcasted_iota(jnp.int32, (T, E), 1)
    NEG = jnp.float32(-1e30)
    m1 = jnp.max(logits, axis=1, keepdims=True)
    i1 = jnp.min(jnp.where(logits == m1, iota_e, E), axis=1, keepdims=True)
    sel1 = iota_e == i1
    masked = jnp.where(sel1, NEG, logits)
    m2 = jnp.max(masked, axis=1, keepdims=True)
    i2 = jnp.min(jnp.where(masked == m2, iota_e, E), axis=1, keepdims=True)
    sel2 = iota_e == i2
    wa = 1.0 / (1.0 + jnp.exp(m2 - m1))
    wb = 1.0 / (1.0 + jnp.exp(m1 - m2))

    cnt = sel1.astype(jnp.int32) + sel2.astype(jnp.int32)
    C = _cumsum0(cnt)
    tot = jnp.sum(cnt, axis=0, keepdims=True)
    lane16 = jax.lax.broadcasted_iota(jnp.int32, (1, 16), 1)
    cnt16 = jnp.where(lane16 < E,
                      jnp.pad(tot, ((0, 0), (0, 8))), 0)
    off16 = _cumsum1_excl(cnt16)
    offc = off16[:, :E]

    rank0 = jnp.sum(jnp.where(sel1, C, 0), axis=1, keepdims=True)
    rank1 = jnp.sum(jnp.where(sel2, C, 0), axis=1, keepdims=True)
    base0 = jnp.sum(jnp.where(sel1, offc, 0), axis=1, keepdims=True)
    base1 = jnp.sum(jnp.where(sel2, offc, 0), axis=1, keepdims=True)
    pos0 = base0 + rank0
    pos1 = base1 + rank1

    pos_ref[...] = jnp.concatenate([pos0, pos1], axis=1)
    w_ref[...] = jnp.concatenate([wa, wb], axis=1)
    off_ref[...] = jnp.broadcast_to(off16, (8, 16))

    ends = offc + tot
    bcol = jax.lax.broadcasted_iota(jnp.int32, (NB, 1), 0) * BM
    blk_lo = jnp.sum((jnp.broadcast_to(ends, (NB, E)) <= bcol)
                     .astype(jnp.int32), axis=1, keepdims=True)
    blk_hi = jnp.sum((jnp.broadcast_to(ends, (NB, E)) <= bcol + (BM - 1))
                     .astype(jnp.int32), axis=1, keepdims=True)
    lo_ref[...] = blk_lo
    hi_ref[...] = blk_hi


def _router(x, Wg, bg):
    return pl.pallas_call(
        _router_body,
        out_shape=(
            jax.ShapeDtypeStruct((T, K), jnp.int32),
            jax.ShapeDtypeStruct((T, K), jnp.float32),
            jax.ShapeDtypeStruct((8, 16), jnp.int32),
            jax.ShapeDtypeStruct((NB, 1), jnp.int32),
            jax.ShapeDtypeStruct((NB, 1), jnp.int32),
        ),
    )(x, Wg, bg.reshape(1, E))


def _gelu_tanh(v):
    c = jnp.float32(0.7978845608028654)
    return 0.5 * v * (1.0 + jnp.tanh(c * (v + 0.044715 * v * v * v)))


def _gmm_body(off_ref, lo_ref, hi_ref,
              xs_ref, w1g_ref, w1u_ref, w2_ref, ws_ref, out_ref):
    b = pl.program_id(0)
    e = pl.program_id(1)
    lo_e = lo_ref[b]
    hi_e = hi_ref[b]

    @pl.when(e == lo_e)
    def _zero():
        out_ref[...] = jnp.zeros_like(out_ref)

    @pl.when(jnp.logical_and(e >= lo_e, e <= hi_e))
    def _compute():
        row0 = jnp.maximum(off_ref[e] - b * BM, 0)
        row1 = jnp.minimum(off_ref[e + 1] - b * BM, BM)
        x = xs_ref[...]
        g = jax.lax.dot_general(
            x, w1g_ref[0], (((1,), (1,)), ((), ())),
            preferred_element_type=jnp.float32)
        u = jax.lax.dot_general(
            x, w1u_ref[0], (((1,), (1,)), ((), ())),
            preferred_element_type=jnp.float32)
        h = _gelu_tanh(g) * u * ws_ref[...]
        o = jax.lax.dot_general(
            h, w2_ref[0], (((1,), (1,)), ((), ())),
            preferred_element_type=jnp.float32)
        rows = jax.lax.broadcasted_iota(jnp.int32, (BM, 1), 0)
        m = jnp.logical_and(rows >= row0, rows < row1)
        out_ref[...] = out_ref[...] + jnp.where(m, o, 0.0)


def _gmm(off, lo, hi, xs, w1g, w1u, w2, ws):
    eclamp = lambda e, lo_ref, hi_ref, b: jnp.clip(e, lo_ref[b], hi_ref[b])
    grid_spec = pltpu.PrefetchScalarGridSpec(
        num_scalar_prefetch=3,
        grid=(NB, E),
        in_specs=[
            pl.BlockSpec((BM, H), lambda b, e, *_: (b, 0)),
            pl.BlockSpec((1, F, H),
                         lambda b, e, o, l, h: (eclamp(e, l, h, b), 0, 0)),
            pl.BlockSpec((1, F, H),
                         lambda b, e, o, l, h: (eclamp(e, l, h, b), 0, 0)),
            pl.BlockSpec((1, H, F),
                         lambda b, e, o, l, h: (eclamp(e, l, h, b), 0, 0)),
            pl.BlockSpec((BM, 1), lambda b, e, *_: (b, 0)),
        ],
        out_specs=pl.BlockSpec((BM, H), lambda b, e, *_: (b, 0)),
    )
    return pl.pallas_call(
        _gmm_body,
        grid_spec=grid_spec,
        out_shape=jax.ShapeDtypeStruct((P, H), jnp.float32),
        compiler_params=pltpu.CompilerParams(
            dimension_semantics=("arbitrary", "arbitrary")),
    )(off, lo, hi, xs, w1g, w1u, w2, ws)


def kernel(x, Wg, bg, w1, w2):
    pos, wts, off16, blk_lo, blk_hi = _router(x, Wg, bg)
    off = off16[0]
    lo = blk_lo.reshape(NB)
    hi = blk_hi.reshape(NB)

    pos_flat = pos.reshape(P)
    inv = jnp.zeros((P,), jnp.int32).at[pos_flat].set(
        jnp.arange(P, dtype=jnp.int32) // K)
    xs = x[inv]
    ws = jnp.zeros((P,), jnp.float32).at[pos_flat].set(wts.reshape(P))

    w1g = w1[:, :F, :]
    w1u = w1[:, F:, :]
    out_sorted = _gmm(off, lo, hi, xs, w1g, w1u, w2, ws.reshape(P, 1))

    y = out_sorted[pos[:, 0]] + out_sorted[pos[:, 1]]
    return y


# speedup vs baseline: 1.1191x; 1.0758x over previous
import functools

import jax
import jax.numpy as jnp
from jax import lax
from jax.experimental import pallas as pl
from jax.experimental.pallas import tpu as pltpu
from jax.experimental.pallas import tpu_sc as plsc

E = 8
K = 2
H = 2048
F = 1024
T = 2048
P = T * K
BM = 256
NB = P // BM


def _cumsum1_excl(a):
    n = a.shape[1]
    z = jnp.zeros((a.shape[0], 1), a.dtype)
    a = jnp.concatenate([z, a[:, :-1]], axis=1)
    s = 1
    while s < n:
        zz = jnp.zeros((a.shape[0], s), a.dtype)
        a = a + jnp.concatenate([zz, a[:, :-s]], axis=1)
        s *= 2
    return a


def _cumsum0(a):
    n = a.shape[0]
    z = jnp.zeros((1,) + a.shape[1:], a.dtype)
    a = jnp.concatenate([z, a[:-1]], axis=0)
    s = 1
    while s < n:
        zz = jnp.zeros((s,) + a.shape[1:], a.dtype)
        a = a + jnp.concatenate([zz, a[:-s]], axis=0)
        s *= 2
    return a


def _router_body(x_ref, wg_ref, bg_ref,
                 pos_ref, w_ref, off_ref, lo_ref, hi_ref):
    x = x_ref[...]
    wg = wg_ref[...]
    logits = jax.lax.dot_general(
        x, wg, (((1,), (1,)), ((), ())),
        preferred_element_type=jnp.float32)
    logits = logits + bg_ref[...]

    iota_e = jax.lax.broadcasted_iota(jnp.int32, (T, E), 1)
    NEG = jnp.float32(-1e30)
    m1 = jnp.max(logits, axis=1, keepdims=True)
    i1 = jnp.min(jnp.where(logits == m1, iota_e, E), axis=1, keepdims=True)
    sel1 = iota_e == i1
    masked = jnp.where(sel1, NEG, logits)
    m2 = jnp.max(masked, axis=1, keepdims=True)
    i2 = jnp.min(jnp.where(masked == m2, iota_e, E), axis=1, keepdims=True)
    sel2 = iota_e == i2
    wa = 1.0 / (1.0 + jnp.exp(m2 - m1))
    wb = 1.0 / (1.0 + jnp.exp(m1 - m2))

    cnt = sel1.astype(jnp.int32) + sel2.astype(jnp.int32)
    C = _cumsum0(cnt)
    tot = jnp.sum(cnt, axis=0, keepdims=True)
    lane16 = jax.lax.broadcasted_iota(jnp.int32, (1, 16), 1)
    cnt16 = jnp.where(lane16 < E,
                      jnp.pad(tot, ((0, 0), (0, 8))), 0)
    off16 = _cumsum1_excl(cnt16)
    offc = off16[:, :E]

    rank0 = jnp.sum(jnp.where(sel1, C, 0), axis=1, keepdims=True)
    rank1 = jnp.sum(jnp.where(sel2, C, 0), axis=1, keepdims=True)
    base0 = jnp.sum(jnp.where(sel1, offc, 0), axis=1, keepdims=True)
    base1 = jnp.sum(jnp.where(sel2, offc, 0), axis=1, keepdims=True)
    pos0 = base0 + rank0
    pos1 = base1 + rank1

    pos_ref[...] = jnp.concatenate([pos0, pos1], axis=1)
    w_ref[...] = jnp.concatenate([wa, wb], axis=1)
    off_ref[...] = jnp.broadcast_to(off16, (8, 16))

    ends = offc + tot
    bcol = jax.lax.broadcasted_iota(jnp.int32, (NB, 1), 0) * BM
    blk_lo = jnp.sum((jnp.broadcast_to(ends, (NB, E)) <= bcol)
                     .astype(jnp.int32), axis=1, keepdims=True)
    blk_hi = jnp.sum((jnp.broadcast_to(ends, (NB, E)) <= bcol + (BM - 1))
                     .astype(jnp.int32), axis=1, keepdims=True)
    lo_ref[...] = blk_lo
    hi_ref[...] = blk_hi


def _router(x, Wg, bg):
    return pl.pallas_call(
        _router_body,
        out_shape=(
            jax.ShapeDtypeStruct((T, K), jnp.int32),
            jax.ShapeDtypeStruct((T, K), jnp.float32),
            jax.ShapeDtypeStruct((8, 16), jnp.int32),
            jax.ShapeDtypeStruct((NB, 1), jnp.int32),
            jax.ShapeDtypeStruct((NB, 1), jnp.int32),
        ),
    )(x, Wg, bg.reshape(1, E))


def _gelu_tanh(v):
    c = jnp.float32(0.7978845608028654)
    return 0.5 * v * (1.0 + jnp.tanh(c * (v + 0.044715 * v * v * v)))


def _gmm_body(off_ref, lo_ref, hi_ref,
              xs_ref, w1g_ref, w1u_ref, w2_ref, out_ref):
    b = pl.program_id(0)
    e = pl.program_id(1)
    lo_e = lo_ref[b]
    hi_e = hi_ref[b]

    @pl.when(e == lo_e)
    def _zero():
        out_ref[...] = jnp.zeros_like(out_ref)

    @pl.when(jnp.logical_and(e >= lo_e, e <= hi_e))
    def _compute():
        row0 = jnp.maximum(off_ref[e] - b * BM, 0)
        row1 = jnp.minimum(off_ref[e + 1] - b * BM, BM)
        x = xs_ref[...]
        g = jax.lax.dot_general(
            x, w1g_ref[0], (((1,), (1,)), ((), ())),
            preferred_element_type=jnp.float32)
        u = jax.lax.dot_general(
            x, w1u_ref[0], (((1,), (1,)), ((), ())),
            preferred_element_type=jnp.float32)
        h = _gelu_tanh(g) * u
        o = jax.lax.dot_general(
            h, w2_ref[0], (((1,), (1,)), ((), ())),
            preferred_element_type=jnp.float32)
        rows = jax.lax.broadcasted_iota(jnp.int32, (BM, 1), 0)
        m = jnp.logical_and(rows >= row0, rows < row1)
        out_ref[...] = out_ref[...] + jnp.where(m, o, 0.0)


def _gmm(off, lo, hi, xs, w1g, w1u, w2):
    eclamp = lambda e, lo_ref, hi_ref, b: jnp.clip(e, lo_ref[b], hi_ref[b])
    grid_spec = pltpu.PrefetchScalarGridSpec(
        num_scalar_prefetch=3,
        grid=(NB, E),
        in_specs=[
            pl.BlockSpec((BM, H), lambda b, e, *_: (b, 0)),
            pl.BlockSpec((1, F, H),
                         lambda b, e, o, l, h: (eclamp(e, l, h, b), 0, 0)),
            pl.BlockSpec((1, F, H),
                         lambda b, e, o, l, h: (eclamp(e, l, h, b), 0, 0)),
            pl.BlockSpec((1, H, F),
                         lambda b, e, o, l, h: (eclamp(e, l, h, b), 0, 0)),
        ],
        out_specs=pl.BlockSpec((BM, H), lambda b, e, *_: (b, 0)),
    )
    return pl.pallas_call(
        _gmm_body,
        grid_spec=grid_spec,
        out_shape=jax.ShapeDtypeStruct((P, H), jnp.float32),
        compiler_params=pltpu.CompilerParams(
            dimension_semantics=("arbitrary", "arbitrary")),
    )(off, lo, hi, xs, w1g, w1u, w2)


_NW = 32
_TPW = T // _NW
_CH = 16


def _worker_id():
    return lax.axis_index("s") * 2 + lax.axis_index("c")


def _dispatch_body(x_hbm, pos0_hbm, pos1_hbm, xs_hbm, xbuf, idx0, idx1):
    w = _worker_id()
    for c in range(_TPW // _CH):
        base = w * _TPW + c * _CH
        pltpu.sync_copy(pos0_hbm.at[pl.ds(base, _CH)], idx0)
        pltpu.sync_copy(pos1_hbm.at[pl.ds(base, _CH)], idx1)
        pltpu.sync_copy(x_hbm.at[pl.ds(base, _CH)], xbuf)
        pltpu.sync_copy(xbuf, xs_hbm.at[idx0])
        pltpu.sync_copy(xbuf, xs_hbm.at[idx1])


def _dispatch(x, pos0, pos1):
    mesh = plsc.VectorSubcoreMesh(core_axis_name="c", subcore_axis_name="s")
    f = functools.partial(
        pl.kernel,
        out_type=jax.ShapeDtypeStruct((P, H), jnp.float32),
        mesh=mesh,
        scratch_types=[
            pltpu.VMEM((_CH, H), jnp.float32),
            pltpu.VMEM((_CH,), jnp.int32),
            pltpu.VMEM((_CH,), jnp.int32),
        ],
    )(_dispatch_body)
    return f(x, pos0, pos1)


def _combine_body(os_hbm, pos0_hbm, pos1_hbm, w0_hbm, w1_hbm, y_hbm,
                  buf0, buf1, idx0, idx1, widx, wb0, wb1):
    w = _worker_id()
    for c in range(_TPW // _CH):
        base = w * _TPW + c * _CH
        pltpu.sync_copy(pos0_hbm.at[pl.ds(base, _CH)], idx0)
        pltpu.sync_copy(pos1_hbm.at[pl.ds(base, _CH)], idx1)
        for r in range(_CH):
            widx[pl.ds(r * 16, 16)] = jnp.full((16,), base + r, jnp.int32)
        pltpu.sync_copy(w0_hbm.at[widx], wb0)
        pltpu.sync_copy(w1_hbm.at[widx], wb1)
        pltpu.sync_copy(os_hbm.at[idx0], buf0)
        pltpu.sync_copy(os_hbm.at[idx1], buf1)
        for r in range(_CH):
            wa = wb0[pl.ds(r * 16, 16)]
            wb = wb1[pl.ds(r * 16, 16)]

            def body_fn(j, carry, r=r, wa=wa, wb=wb):
                s = j * 16
                buf0[r, pl.ds(s, 16)] = (wa * buf0[r, pl.ds(s, 16)]
                                         + wb * buf1[r, pl.ds(s, 16)])
                return carry

            lax.fori_loop(0, H // 16, body_fn, 0, unroll=8)
        pltpu.sync_copy(buf0, y_hbm.at[pl.ds(base, _CH)])


def _combine(os, pos0, pos1, w0, w1):
    mesh = plsc.VectorSubcoreMesh(core_axis_name="c", subcore_axis_name="s")
    f = functools.partial(
        pl.kernel,
        out_type=jax.ShapeDtypeStruct((T, H), jnp.float32),
        mesh=mesh,
        scratch_types=[
            pltpu.VMEM((_CH, H), jnp.float32),
            pltpu.VMEM((_CH, H), jnp.float32),
            pltpu.VMEM((_CH,), jnp.int32),
            pltpu.VMEM((_CH,), jnp.int32),
            pltpu.VMEM((_CH * 16,), jnp.int32),
            pltpu.VMEM((_CH * 16,), jnp.float32),
            pltpu.VMEM((_CH * 16,), jnp.float32),
        ],
    )(_combine_body)
    return f(os, pos0, pos1, w0, w1)


def kernel(x, Wg, bg, w1, w2):
    pos, wts, off16, blk_lo, blk_hi = _router(x, Wg, bg)
    off = off16[0]
    lo = blk_lo.reshape(NB)
    hi = blk_hi.reshape(NB)
    pos0 = pos[:, 0]
    pos1 = pos[:, 1]

    xs = _dispatch(x, pos0, pos1)

    w1g = w1[:, :F, :]
    w1u = w1[:, F:, :]
    out_sorted = _gmm(off, lo, hi, xs, w1g, w1u, w2)

    y = _combine(out_sorted, pos0, pos1, wts[:, 0], wts[:, 1])
    return y
